# baseline (speedup 1.0000x reference)
import functools

import jax
import jax.numpy as jnp
from jax import lax
from jax.experimental import pallas as pl
from jax.experimental.pallas import tpu as pltpu
from jax.experimental.pallas import tpu_sc as plsc

S = 2048
E = 1024
N = 8
H = 4096
K = 2
P = S * K
MBLK = 128
PPAD = P + N * MBLK
NB = PPAD // MBLK
TBLK = 256
NC = 2
NS = 16
NW = NC * NS
LANES = 16
TOK_R = S // NW

_SC_MESH = plsc.VectorSubcoreMesh(core_axis_name="c", subcore_axis_name="s")
_SC_PARAMS = pltpu.CompilerParams(needs_layout_passes=False)


def _router_body(x_ref, rw1_ref, rb1_ref, rw2_ref, rb2_ref,
                 xbf_ref, wT_ref, posT_ref, bexp_ref,
                 ep_s, rk_s, w_s, carry_ref):
    t = pl.program_id(0)
    nblocks = pl.num_programs(0)
    xbf = x_ref[...].astype(jnp.bfloat16)
    xbf_ref[...] = xbf
    h = jnp.dot(xbf, rw1_ref[...].astype(jnp.bfloat16),
                preferred_element_type=jnp.float32) + rb1_ref[...]
    h = jnp.maximum(h, 0.0).astype(jnp.bfloat16)
    s = jnp.dot(h, rw2_ref[...].astype(jnp.bfloat16),
                preferred_element_type=jnp.float32) + rb2_ref[...]
    lane = lax.broadcasted_iota(jnp.int32, s.shape, 1)
    m1 = jnp.max(s, axis=1, keepdims=True)
    a1 = jnp.min(jnp.where(s == m1, lane, N), axis=1, keepdims=True)
    sm = jnp.where(lane == a1, -jnp.inf, s)
    m2 = jnp.max(sm, axis=1, keepdims=True)
    a2 = jnp.min(jnp.where(sm == m2, lane, N), axis=1, keepdims=True)
    e2 = jnp.exp(m2 - m1)
    w1 = 1.0 / (1.0 + e2)
    w2 = e2 / (1.0 + e2)
    w_s[pl.ds(t * TBLK, TBLK), :] = jnp.concatenate([w1, w2], axis=1)

    oh = (lane == a1).astype(jnp.float32) + (lane == a2).astype(jnp.float32)
    r_i = lax.broadcasted_iota(jnp.int32, (TBLK, TBLK), 0)
    c_i = lax.broadcasted_iota(jnp.int32, (TBLK, TBLK), 1)
    tri = (c_i < r_i).astype(jnp.bfloat16)
    cum = jnp.dot(tri, oh.astype(jnp.bfloat16),
                  preferred_element_type=jnp.float32)

    @pl.when(t == 0)
    def _():
        carry_ref[...] = jnp.zeros_like(carry_ref)

    carry = carry_ref[...]
    cumg = cum + carry
    rank1 = jnp.sum(jnp.where(lane == a1, cumg, 0.0), axis=1, keepdims=True)
    rank2 = jnp.sum(jnp.where(lane == a2, cumg, 0.0), axis=1, keepdims=True)
    ep_s[pl.ds(t * TBLK, TBLK), :] = jnp.concatenate([a1, a2], axis=1)
    rk_s[pl.ds(t * TBLK, TBLK), :] = jnp.concatenate(
        [rank1, rank2], axis=1).astype(jnp.int32)
    carry_ref[...] = carry + jnp.sum(oh, axis=0, keepdims=True)

    @pl.when(t == nblocks - 1)
    def _():
        cnt = carry_ref[...]
        padded = jnp.floor((cnt + (MBLK - 1)) / MBLK) * MBLK
        ui = lax.broadcasted_iota(jnp.int32, (N, N), 0)
        uj = lax.broadcasted_iota(jnp.int32, (N, N), 1)
        offs = jnp.dot(padded.astype(jnp.bfloat16),
                       (ui < uj).astype(jnp.bfloat16),
                       preferred_element_type=jnp.float32)
        offsi = offs + padded
        lane8 = lax.broadcasted_iota(jnp.int32, (1, N), 1)
        ep = ep_s[...]
        rk = rk_s[...].astype(jnp.float32)
        posf = rk
        for e in range(N):
            off_e = jnp.sum(jnp.where(lane8 == e, offs, 0.0))
            posf = posf + jnp.where(ep == e, off_e, 0.0)
        posT_ref[...] = jnp.transpose(posf).astype(jnp.int32)
        wT_ref[...] = jnp.transpose(w_s[...])
        lane128 = lax.broadcasted_iota(jnp.int32, (1, 128), 1)
        row = (lane128 * MBLK).astype(jnp.float32)
        acc = jnp.zeros((1, 128), jnp.int32)
        for e in range(N):
            offi_e = jnp.sum(jnp.where(lane8 == e, offsi, 0.0))
            acc = acc + jnp.where(row >= offi_e, 1, 0)
        bexp_ref[...] = jnp.minimum(acc, N - 1).reshape(128)


HC = 2


def _gmm_body(bexp_ref, posT_ref, wT_ref, xb_ref,
              ew1_ref, eb1_ref, ew2_ref, eb2_ref, o_ref, xs_s, ws_s):
    b = pl.program_id(0)
    c = pl.program_id(1)

    @pl.when(c == 0)
    def _():
        slot = lax.broadcasted_iota(jnp.int32, (MBLK, S), 0) + b * MBLK
        p0 = posT_ref[0:1, :]
        p1 = posT_ref[1:2, :]
        sel0 = slot == p0
        sel1 = slot == p1
        perm = (sel0 | sel1).astype(jnp.bfloat16)
        xs_s[...] = jnp.dot(perm, xb_ref[...],
                            preferred_element_type=jnp.float32
                            ).astype(jnp.bfloat16)
        ws_s[...] = jnp.sum(jnp.where(sel0, wT_ref[0:1, :], 0.0) +
                            jnp.where(sel1, wT_ref[1:2, :], 0.0),
                            axis=1, keepdims=True)

    xs = xs_s[...]
    ws = ws_s[...]
    h = jnp.dot(xs, ew1_ref[0].astype(jnp.bfloat16),
                preferred_element_type=jnp.float32) + eb1_ref[0]
    h = jnp.maximum(h, 0.0).astype(jnp.bfloat16)
    yc = jnp.dot(h, ew2_ref[0].astype(jnp.bfloat16),
                 preferred_element_type=jnp.float32)

    @pl.when(c == 0)
    def _():
        o_ref[...] = (yc + eb2_ref[0]) * ws

    @pl.when(c > 0)
    def _():
        o_ref[...] += yc * ws


def _combine_sc_body(ys_hbm, posT_hbm, out_hbm, idx0_v, idx1_v, rows_a,
                     rows_b, sem):
    cid = lax.axis_index("c")
    sid = lax.axis_index("s")
    wid = sid * NC + cid
    tbase = wid * TOK_R
    pltpu.sync_copy(posT_hbm.at[0, pl.ds(tbase, TOK_R)], idx0_v)
    pltpu.sync_copy(posT_hbm.at[1, pl.ds(tbase, TOK_R)], idx1_v)
    half = TOK_R // 2
    for c in range(2):
        cp1 = pltpu.async_copy(
            ys_hbm.at[idx0_v.at[pl.ds(c * half, half)]], rows_a, sem)
        cp2 = pltpu.async_copy(
            ys_hbm.at[idx1_v.at[pl.ds(c * half, half)]], rows_b, sem)
        cp1.wait()
        cp2.wait()

        @pl.loop(0, half)
        def _(u):
            @pl.loop(0, E, step=LANES)
            def _(j):
                rows_a[u, pl.ds(j, LANES)] = (rows_a[u, pl.ds(j, LANES)] +
                                              rows_b[u, pl.ds(j, LANES)])

        pltpu.sync_copy(rows_a, out_hbm.at[pl.ds(tbase + c * half, half)])


@jax.jit
def kernel(inputs, rw1, rb1, rw2, rb2, ew1, eb1, ew2, eb2):
    x2 = inputs.reshape(S, E)

    xb, wT, posT, bexp = pl.pallas_call(
        _router_body,
        grid=(S // TBLK,),
        in_specs=[
            pl.BlockSpec((TBLK, E), lambda i: (i, 0)),
            pl.BlockSpec((E, E), lambda i: (0, 0)),
            pl.BlockSpec((E,), lambda i: (0,)),
            pl.BlockSpec((E, N), lambda i: (0, 0)),
            pl.BlockSpec((N,), lambda i: (0,)),
        ],
        out_specs=[
            pl.BlockSpec((TBLK, E), lambda i: (i, 0)),
            pl.BlockSpec((K, S), lambda i: (0, 0)),
            pl.BlockSpec((K, S), lambda i: (0, 0)),
            pl.BlockSpec((128,), lambda i: (0,)),
        ],
        out_shape=[
            jax.ShapeDtypeStruct((S, E), jnp.bfloat16),
            jax.ShapeDtypeStruct((K, S), jnp.float32),
            jax.ShapeDtypeStruct((K, S), jnp.int32),
            jax.ShapeDtypeStruct((128,), jnp.int32),
        ],
        scratch_shapes=[
            pltpu.VMEM((S, K), jnp.int32),
            pltpu.VMEM((S, K), jnp.int32),
            pltpu.VMEM((S, K), jnp.float32),
            pltpu.VMEM((1, N), jnp.float32),
        ],
    )(x2, rw1, rb1, rw2, rb2)

    grid_spec = pltpu.PrefetchScalarGridSpec(
        num_scalar_prefetch=1,
        grid=(NB, HC),
        in_specs=[
            pl.BlockSpec((K, S), lambda b, c, s: (0, 0)),
            pl.BlockSpec((K, S), lambda b, c, s: (0, 0)),
            pl.BlockSpec((S, E), lambda b, c, s: (0, 0)),
            pl.BlockSpec((1, E, H // HC), lambda b, c, s: (s[b], 0, c)),
            pl.BlockSpec((1, 1, H // HC), lambda b, c, s: (s[b], 0, c)),
            pl.BlockSpec((1, H // HC, E), lambda b, c, s: (s[b], c, 0)),
            pl.BlockSpec((1, 1, E), lambda b, c, s: (s[b], 0, 0)),
        ],
        out_specs=pl.BlockSpec((MBLK, E), lambda b, c, s: (b, 0)),
        scratch_shapes=[
            pltpu.VMEM((MBLK, E), jnp.bfloat16),
            pltpu.VMEM((MBLK, 1), jnp.float32),
        ],
    )
    ys = pl.pallas_call(
        _gmm_body,
        grid_spec=grid_spec,
        out_shape=jax.ShapeDtypeStruct((PPAD, E), jnp.float32),
        compiler_params=pltpu.CompilerParams(
            dimension_semantics=("arbitrary", "arbitrary"),
        ),
    )(bexp, posT, wT, xb, ew1,
      eb1.reshape(N, 1, H), ew2, eb2.reshape(N, 1, E))

    combine_sc = pl.kernel(
        _combine_sc_body,
        out_type=jax.ShapeDtypeStruct((S, E), jnp.float32),
        mesh=_SC_MESH,
        scratch_types=[
            pltpu.VMEM((TOK_R,), jnp.int32),
            pltpu.VMEM((TOK_R,), jnp.int32),
            pltpu.VMEM((TOK_R // 2, E), jnp.float32),
            pltpu.VMEM((TOK_R // 2, E), jnp.float32),
            pltpu.SemaphoreType.DMA,
        ],
        compiler_params=_SC_PARAMS,
    )
    out = combine_sc(ys, posT)
    return out.reshape(1, S, E)


# speedup vs baseline: 1.3952x; 1.3952x over previous
import functools

import jax
import jax.numpy as jnp
from jax import lax
from jax.experimental import pallas as pl
from jax.experimental.pallas import tpu as pltpu
from jax.experimental.pallas import tpu_sc as plsc

S = 2048
E = 1024
N = 8
H = 4096
K = 2
P = S * K
MBLK = 128
PPAD = P + N * MBLK
NB = PPAD // MBLK
TBLK = 256
NC = 2
NS = 16
NW = NC * NS
LANES = 16
TOK_R = S // NW

_SC_MESH = plsc.VectorSubcoreMesh(core_axis_name="c", subcore_axis_name="s")
_SC_PARAMS = pltpu.CompilerParams(needs_layout_passes=False)


WCH = H // 4
RG = 8
CG1 = RG + N * 4
CG2 = CG1 + N * 4


def _router_body(x_ref, rw1_ref, rb1_ref, rw2_ref, rb2_ref,
                 ew1_ref, ew2_ref,
                 xbf_ref, wT_ref, posT_ref, bexp_ref, ew1b_ref, ew2b_ref,
                 ep_s, rk_s, w_s, carry_ref):
    t = pl.program_id(0)
    nblocks = RG

    @pl.when((t >= RG) & (t < CG1))
    def _():
        ew1b_ref[...] = ew1_ref[...].astype(jnp.bfloat16)

    @pl.when(t >= CG1)
    def _():
        ew2b_ref[...] = ew2_ref[...].astype(jnp.bfloat16)

    @pl.when(t < RG)
    def _():
        xbf = x_ref[...].astype(jnp.bfloat16)
        xbf_ref[...] = xbf
        h = jnp.dot(xbf, rw1_ref[...].astype(jnp.bfloat16),
                    preferred_element_type=jnp.float32) + rb1_ref[...]
        h = jnp.maximum(h, 0.0).astype(jnp.bfloat16)
        s = jnp.dot(h, rw2_ref[...].astype(jnp.bfloat16),
                    preferred_element_type=jnp.float32) + rb2_ref[...]
        lane = lax.broadcasted_iota(jnp.int32, s.shape, 1)
        m1 = jnp.max(s, axis=1, keepdims=True)
        a1 = jnp.min(jnp.where(s == m1, lane, N), axis=1, keepdims=True)
        sm = jnp.where(lane == a1, -jnp.inf, s)
        m2 = jnp.max(sm, axis=1, keepdims=True)
        a2 = jnp.min(jnp.where(sm == m2, lane, N), axis=1, keepdims=True)
        e2 = jnp.exp(m2 - m1)
        w1 = 1.0 / (1.0 + e2)
        w2 = e2 / (1.0 + e2)
        w_s[pl.ds(t * TBLK, TBLK), :] = jnp.concatenate([w1, w2], axis=1)

        oh = ((lane == a1).astype(jnp.float32) +
              (lane == a2).astype(jnp.float32))
        r_i = lax.broadcasted_iota(jnp.int32, (TBLK, TBLK), 0)
        c_i = lax.broadcasted_iota(jnp.int32, (TBLK, TBLK), 1)
        tri = (c_i < r_i).astype(jnp.bfloat16)
        cum = jnp.dot(tri, oh.astype(jnp.bfloat16),
                      preferred_element_type=jnp.float32)

        @pl.when(t == 0)
        def _():
            carry_ref[...] = jnp.zeros_like(carry_ref)

        carry = carry_ref[...]
        cumg = cum + carry
        rank1 = jnp.sum(jnp.where(lane == a1, cumg, 0.0), axis=1,
                        keepdims=True)
        rank2 = jnp.sum(jnp.where(lane == a2, cumg, 0.0), axis=1,
                        keepdims=True)
        ep_s[pl.ds(t * TBLK, TBLK), :] = jnp.concatenate([a1, a2], axis=1)
        rk_s[pl.ds(t * TBLK, TBLK), :] = jnp.concatenate(
            [rank1, rank2], axis=1).astype(jnp.int32)
        carry_ref[...] = carry + jnp.sum(oh, axis=0, keepdims=True)

    @pl.when(t == nblocks - 1)
    def _():
        cnt = carry_ref[...]
        padded = jnp.floor((cnt + (MBLK - 1)) / MBLK) * MBLK
        ui = lax.broadcasted_iota(jnp.int32, (N, N), 0)
        uj = lax.broadcasted_iota(jnp.int32, (N, N), 1)
        offs = jnp.dot(padded.astype(jnp.bfloat16),
                       (ui < uj).astype(jnp.bfloat16),
                       preferred_element_type=jnp.float32)
        offsi = offs + padded
        lane8 = lax.broadcasted_iota(jnp.int32, (1, N), 1)
        ep = ep_s[...]
        rk = rk_s[...].astype(jnp.float32)
        posf = rk
        for e in range(N):
            off_e = jnp.sum(jnp.where(lane8 == e, offs, 0.0))
            posf = posf + jnp.where(ep == e, off_e, 0.0)
        posT_ref[...] = jnp.transpose(posf).astype(jnp.int32)
        wT_ref[...] = jnp.transpose(w_s[...])
        lane128 = lax.broadcasted_iota(jnp.int32, (1, 128), 1)
        row = (lane128 * MBLK).astype(jnp.float32)
        acc = jnp.zeros((1, 128), jnp.int32)
        for e in range(N):
            offi_e = jnp.sum(jnp.where(lane8 == e, offsi, 0.0))
            acc = acc + jnp.where(row >= offi_e, 1, 0)
        bexp_ref[...] = jnp.minimum(acc, N - 1).reshape(128)


def _gmm_body(bexp_ref, posT_ref, wT_ref, xb_ref,
              ew1_ref, eb1_ref, ew2_ref, eb2_ref, o_ref):
    b = pl.program_id(0)
    slot = lax.broadcasted_iota(jnp.int32, (MBLK, S), 0) + b * MBLK
    p0 = posT_ref[0:1, :]
    p1 = posT_ref[1:2, :]
    sel0 = slot == p0
    sel1 = slot == p1
    perm = (sel0 | sel1).astype(jnp.bfloat16)
    xs = jnp.dot(perm, xb_ref[...],
                 preferred_element_type=jnp.float32).astype(jnp.bfloat16)
    ws = jnp.sum(jnp.where(sel0, wT_ref[0:1, :], 0.0) +
                 jnp.where(sel1, wT_ref[1:2, :], 0.0),
                 axis=1, keepdims=True)
    h = jnp.dot(xs, ew1_ref[0], preferred_element_type=jnp.float32) + eb1_ref[0]
    h = jnp.maximum(h, 0.0).astype(jnp.bfloat16)
    y = jnp.dot(h, ew2_ref[0], preferred_element_type=jnp.float32) + eb2_ref[0]
    o_ref[...] = y * ws


def _combine_sc_body(ys_hbm, posT_hbm, out_hbm, idx0_v, idx1_v, rows_a,
                     rows_b, sem):
    cid = lax.axis_index("c")
    sid = lax.axis_index("s")
    wid = sid * NC + cid
    tbase = wid * TOK_R
    pltpu.sync_copy(posT_hbm.at[0, pl.ds(tbase, TOK_R)], idx0_v)
    pltpu.sync_copy(posT_hbm.at[1, pl.ds(tbase, TOK_R)], idx1_v)
    half = TOK_R // 2
    for c in range(2):
        cp1 = pltpu.async_copy(
            ys_hbm.at[idx0_v.at[pl.ds(c * half, half)]], rows_a, sem)
        cp2 = pltpu.async_copy(
            ys_hbm.at[idx1_v.at[pl.ds(c * half, half)]], rows_b, sem)
        cp1.wait()
        cp2.wait()

        @pl.loop(0, half)
        def _(u):
            @pl.loop(0, E, step=LANES)
            def _(j):
                rows_a[u, pl.ds(j, LANES)] = (rows_a[u, pl.ds(j, LANES)] +
                                              rows_b[u, pl.ds(j, LANES)])

        pltpu.sync_copy(rows_a, out_hbm.at[pl.ds(tbase + c * half, half)])


@jax.jit
def kernel(inputs, rw1, rb1, rw2, rb2, ew1, eb1, ew2, eb2):
    x2 = inputs.reshape(S, E)

    def _xmap(t):
        return (jnp.minimum(t, RG - 1), 0)

    def _w1map(t):
        j = jnp.clip(t - RG, 0, N * 4 - 1)
        return (j // 4, 0, j % 4)

    def _w2map(t):
        j = jnp.clip(t - CG1, 0, N * 4 - 1)
        return (j // 4, j % 4, 0)

    xb, wT, posT, bexp, ew1b, ew2b = pl.pallas_call(
        _router_body,
        grid=(CG2,),
        in_specs=[
            pl.BlockSpec((TBLK, E), _xmap),
            pl.BlockSpec((E, E), lambda i: (0, 0)),
            pl.BlockSpec((E,), lambda i: (0,)),
            pl.BlockSpec((E, N), lambda i: (0, 0)),
            pl.BlockSpec((N,), lambda i: (0,)),
            pl.BlockSpec((1, E, WCH), _w1map),
            pl.BlockSpec((1, WCH, E), _w2map),
        ],
        out_specs=[
            pl.BlockSpec((TBLK, E), _xmap),
            pl.BlockSpec((K, S), lambda i: (0, 0)),
            pl.BlockSpec((K, S), lambda i: (0, 0)),
            pl.BlockSpec((128,), lambda i: (0,)),
            pl.BlockSpec((1, E, WCH), _w1map),
            pl.BlockSpec((1, WCH, E), _w2map),
        ],
        out_shape=[
            jax.ShapeDtypeStruct((S, E), jnp.bfloat16),
            jax.ShapeDtypeStruct((K, S), jnp.float32),
            jax.ShapeDtypeStruct((K, S), jnp.int32),
            jax.ShapeDtypeStruct((128,), jnp.int32),
            jax.ShapeDtypeStruct((N, E, H), jnp.bfloat16),
            jax.ShapeDtypeStruct((N, H, E), jnp.bfloat16),
        ],
        scratch_shapes=[
            pltpu.VMEM((S, K), jnp.int32),
            pltpu.VMEM((S, K), jnp.int32),
            pltpu.VMEM((S, K), jnp.float32),
            pltpu.VMEM((1, N), jnp.float32),
        ],
        compiler_params=pltpu.CompilerParams(
            dimension_semantics=("arbitrary",),
        ),
    )(x2, rw1, rb1, rw2, rb2, ew1, ew2)

    grid_spec = pltpu.PrefetchScalarGridSpec(
        num_scalar_prefetch=1,
        grid=(NB,),
        in_specs=[
            pl.BlockSpec((K, S), lambda b, s: (0, 0)),
            pl.BlockSpec((K, S), lambda b, s: (0, 0)),
            pl.BlockSpec((S, E), lambda b, s: (0, 0)),
            pl.BlockSpec((1, E, H), lambda b, s: (s[b], 0, 0)),
            pl.BlockSpec((1, 1, H), lambda b, s: (s[b], 0, 0)),
            pl.BlockSpec((1, H, E), lambda b, s: (s[b], 0, 0)),
            pl.BlockSpec((1, 1, E), lambda b, s: (s[b], 0, 0)),
        ],
        out_specs=pl.BlockSpec((MBLK, E), lambda b, s: (b, 0)),
    )
    ys = pl.pallas_call(
        _gmm_body,
        grid_spec=grid_spec,
        out_shape=jax.ShapeDtypeStruct((PPAD, E), jnp.float32),
        compiler_params=pltpu.CompilerParams(
            dimension_semantics=("arbitrary",),
        ),
    )(bexp, posT, wT, xb, ew1b,
      eb1.reshape(N, 1, H), ew2b, eb2.reshape(N, 1, E))

    combine_sc = pl.kernel(
        _combine_sc_body,
        out_type=jax.ShapeDtypeStruct((S, E), jnp.float32),
        mesh=_SC_MESH,
        scratch_types=[
            pltpu.VMEM((TOK_R,), jnp.int32),
            pltpu.VMEM((TOK_R,), jnp.int32),
            pltpu.VMEM((TOK_R // 2, E), jnp.float32),
            pltpu.VMEM((TOK_R // 2, E), jnp.float32),
            pltpu.SemaphoreType.DMA,
        ],
        compiler_params=_SC_PARAMS,
    )
    out = combine_sc(ys, posT)
    return out.reshape(1, S, E)


# speedup vs baseline: 1.6009x; 1.1474x over previous
import functools

import jax
import jax.numpy as jnp
from jax import lax
from jax.experimental import pallas as pl
from jax.experimental.pallas import tpu as pltpu
from jax.experimental.pallas import tpu_sc as plsc

S = 2048
E = 1024
N = 8
H = 4096
K = 2
P = S * K
MBLK = 128
PPAD = P + N * MBLK
NB = PPAD // MBLK
TBLK = 256
NC = 2
NS = 16
NW = NC * NS
LANES = 16
TOK_R = S // NW

_SC_MESH = plsc.VectorSubcoreMesh(core_axis_name="c", subcore_axis_name="s")
_SC_PARAMS = pltpu.CompilerParams(needs_layout_passes=False)


RG = 8


def _router_body(x_ref, rw1_ref, rb1_ref, rw2_ref, rb2_ref,
                 xbf_ref, wT_ref, posT_ref, bexp_ref,
                 ep_s, rk_s, w_s, carry_ref):
    t = pl.program_id(0)
    nblocks = RG

    @pl.when(t < RG)
    def _():
        xbf = x_ref[...].astype(jnp.bfloat16)
        xbf_ref[...] = xbf
        h = jnp.dot(xbf, rw1_ref[...].astype(jnp.bfloat16),
                    preferred_element_type=jnp.float32) + rb1_ref[...]
        h = jnp.maximum(h, 0.0).astype(jnp.bfloat16)
        s = jnp.dot(h, rw2_ref[...].astype(jnp.bfloat16),
                    preferred_element_type=jnp.float32) + rb2_ref[...]
        lane = lax.broadcasted_iota(jnp.int32, s.shape, 1)
        m1 = jnp.max(s, axis=1, keepdims=True)
        a1 = jnp.min(jnp.where(s == m1, lane, N), axis=1, keepdims=True)
        sm = jnp.where(lane == a1, -jnp.inf, s)
        m2 = jnp.max(sm, axis=1, keepdims=True)
        a2 = jnp.min(jnp.where(sm == m2, lane, N), axis=1, keepdims=True)
        e2 = jnp.exp(m2 - m1)
        w1 = 1.0 / (1.0 + e2)
        w2 = e2 / (1.0 + e2)
        w_s[pl.ds(t * TBLK, TBLK), :] = jnp.concatenate([w1, w2], axis=1)

        oh = ((lane == a1).astype(jnp.float32) +
              (lane == a2).astype(jnp.float32))
        r_i = lax.broadcasted_iota(jnp.int32, (TBLK, TBLK), 0)
        c_i = lax.broadcasted_iota(jnp.int32, (TBLK, TBLK), 1)
        tri = (c_i < r_i).astype(jnp.bfloat16)
        cum = jnp.dot(tri, oh.astype(jnp.bfloat16),
                      preferred_element_type=jnp.float32)

        @pl.when(t == 0)
        def _():
            carry_ref[...] = jnp.zeros_like(carry_ref)

        carry = carry_ref[...]
        cumg = cum + carry
        rank1 = jnp.sum(jnp.where(lane == a1, cumg, 0.0), axis=1,
                        keepdims=True)
        rank2 = jnp.sum(jnp.where(lane == a2, cumg, 0.0), axis=1,
                        keepdims=True)
        ep_s[pl.ds(t * TBLK, TBLK), :] = jnp.concatenate([a1, a2], axis=1)
        rk_s[pl.ds(t * TBLK, TBLK), :] = jnp.concatenate(
            [rank1, rank2], axis=1).astype(jnp.int32)
        carry_ref[...] = carry + jnp.sum(oh, axis=0, keepdims=True)

    @pl.when(t == nblocks - 1)
    def _():
        cnt = carry_ref[...]
        padded = jnp.floor((cnt + (MBLK - 1)) / MBLK) * MBLK
        ui = lax.broadcasted_iota(jnp.int32, (N, N), 0)
        uj = lax.broadcasted_iota(jnp.int32, (N, N), 1)
        offs = jnp.dot(padded.astype(jnp.bfloat16),
                       (ui < uj).astype(jnp.bfloat16),
                       preferred_element_type=jnp.float32)
        offsi = offs + padded
        lane8 = lax.broadcasted_iota(jnp.int32, (1, N), 1)
        ep = ep_s[...]
        rk = rk_s[...].astype(jnp.float32)
        posf = rk
        for e in range(N):
            off_e = jnp.sum(jnp.where(lane8 == e, offs, 0.0))
            posf = posf + jnp.where(ep == e, off_e, 0.0)
        posT_ref[...] = jnp.transpose(posf).astype(jnp.int32)
        wT_ref[...] = jnp.transpose(w_s[...])
        lane128 = lax.broadcasted_iota(jnp.int32, (1, 128), 1)
        row = (lane128 * MBLK).astype(jnp.float32)
        acc = jnp.zeros((1, 128), jnp.int32)
        for e in range(N):
            offi_e = jnp.sum(jnp.where(lane8 == e, offsi, 0.0))
            acc = acc + jnp.where(row >= offi_e, 1, 0)
        bexp_ref[...] = jnp.minimum(acc, N - 1).reshape(128)


def _expert_changed(bexp_ref, b):
    prev = bexp_ref[jnp.maximum(b - 1, 0)]
    return jnp.logical_or(b == 0, bexp_ref[b] != prev)


def _mm1_body(bexp_ref, posT_ref, wT_ref, xb_ref, ew1_ref, eb1_ref,
              h_ref, ws_ref, w1bf_s):
    b = pl.program_id(0)

    @pl.when(_expert_changed(bexp_ref, b))
    def _():
        w1bf_s[...] = ew1_ref[0].astype(jnp.bfloat16)

    slot = lax.broadcasted_iota(jnp.int32, (MBLK, S), 0) + b * MBLK
    p0 = posT_ref[0:1, :]
    p1 = posT_ref[1:2, :]
    sel0 = slot == p0
    sel1 = slot == p1
    perm = (sel0 | sel1).astype(jnp.bfloat16)
    xs = jnp.dot(perm, xb_ref[...],
                 preferred_element_type=jnp.float32).astype(jnp.bfloat16)
    ws_ref[...] = jnp.sum(jnp.where(sel0, wT_ref[0:1, :], 0.0) +
                          jnp.where(sel1, wT_ref[1:2, :], 0.0),
                          axis=1, keepdims=True)
    h = jnp.dot(xs, w1bf_s[...],
                preferred_element_type=jnp.float32) + eb1_ref[0]
    h_ref[...] = jnp.maximum(h, 0.0).astype(jnp.bfloat16)


def _mm2_body(bexp_ref, h_ref, ew2_ref, eb2_ref, ws_ref, o_ref, w2bf_s):
    b = pl.program_id(0)

    @pl.when(_expert_changed(bexp_ref, b))
    def _():
        w2bf_s[...] = ew2_ref[0].astype(jnp.bfloat16)

    y = jnp.dot(h_ref[...], w2bf_s[...],
                preferred_element_type=jnp.float32) + eb2_ref[0]
    o_ref[...] = y * ws_ref[...]


def _combine_sc_body(ys_hbm, posT_hbm, out_hbm, idx0_v, idx1_v, rows_a,
                     rows_b, sem):
    cid = lax.axis_index("c")
    sid = lax.axis_index("s")
    wid = sid * NC + cid
    tbase = wid * TOK_R
    pltpu.sync_copy(posT_hbm.at[0, pl.ds(tbase, TOK_R)], idx0_v)
    pltpu.sync_copy(posT_hbm.at[1, pl.ds(tbase, TOK_R)], idx1_v)
    half = TOK_R // 2
    for c in range(2):
        cp1 = pltpu.async_copy(
            ys_hbm.at[idx0_v.at[pl.ds(c * half, half)]], rows_a, sem)
        cp2 = pltpu.async_copy(
            ys_hbm.at[idx1_v.at[pl.ds(c * half, half)]], rows_b, sem)
        cp1.wait()
        cp2.wait()

        @pl.loop(0, half)
        def _(u):
            @pl.loop(0, E, step=LANES)
            def _(j):
                rows_a[u, pl.ds(j, LANES)] = (rows_a[u, pl.ds(j, LANES)] +
                                              rows_b[u, pl.ds(j, LANES)])

        pltpu.sync_copy(rows_a, out_hbm.at[pl.ds(tbase + c * half, half)])


@jax.jit
def kernel(inputs, rw1, rb1, rw2, rb2, ew1, eb1, ew2, eb2):
    x2 = inputs.reshape(S, E)

    xb, wT, posT, bexp = pl.pallas_call(
        _router_body,
        grid=(RG,),
        in_specs=[
            pl.BlockSpec((TBLK, E), lambda i: (i, 0)),
            pl.BlockSpec((E, E), lambda i: (0, 0)),
            pl.BlockSpec((E,), lambda i: (0,)),
            pl.BlockSpec((E, N), lambda i: (0, 0)),
            pl.BlockSpec((N,), lambda i: (0,)),
        ],
        out_specs=[
            pl.BlockSpec((TBLK, E), lambda i: (i, 0)),
            pl.BlockSpec((K, S), lambda i: (0, 0)),
            pl.BlockSpec((K, S), lambda i: (0, 0)),
            pl.BlockSpec((128,), lambda i: (0,)),
        ],
        out_shape=[
            jax.ShapeDtypeStruct((S, E), jnp.bfloat16),
            jax.ShapeDtypeStruct((K, S), jnp.float32),
            jax.ShapeDtypeStruct((K, S), jnp.int32),
            jax.ShapeDtypeStruct((128,), jnp.int32),
        ],
        scratch_shapes=[
            pltpu.VMEM((S, K), jnp.int32),
            pltpu.VMEM((S, K), jnp.int32),
            pltpu.VMEM((S, K), jnp.float32),
            pltpu.VMEM((1, N), jnp.float32),
        ],
    )(x2, rw1, rb1, rw2, rb2)

    mm1_spec = pltpu.PrefetchScalarGridSpec(
        num_scalar_prefetch=1,
        grid=(NB,),
        in_specs=[
            pl.BlockSpec((K, S), lambda b, s: (0, 0)),
            pl.BlockSpec((K, S), lambda b, s: (0, 0)),
            pl.BlockSpec((S, E), lambda b, s: (0, 0)),
            pl.BlockSpec((1, E, H), lambda b, s: (s[b], 0, 0)),
            pl.BlockSpec((1, 1, H), lambda b, s: (s[b], 0, 0)),
        ],
        out_specs=[
            pl.BlockSpec((MBLK, H), lambda b, s: (b, 0)),
            pl.BlockSpec((MBLK, 1), lambda b, s: (b, 0)),
        ],
        scratch_shapes=[pltpu.VMEM((E, H), jnp.bfloat16)],
    )
    hs, wsort = pl.pallas_call(
        _mm1_body,
        grid_spec=mm1_spec,
        out_shape=[
            jax.ShapeDtypeStruct((PPAD, H), jnp.bfloat16),
            jax.ShapeDtypeStruct((PPAD, 1), jnp.float32),
        ],
        compiler_params=pltpu.CompilerParams(
            dimension_semantics=("arbitrary",),
        ),
    )(bexp, posT, wT, xb, ew1, eb1.reshape(N, 1, H))

    mm2_spec = pltpu.PrefetchScalarGridSpec(
        num_scalar_prefetch=1,
        grid=(NB,),
        in_specs=[
            pl.BlockSpec((MBLK, H), lambda b, s: (b, 0)),
            pl.BlockSpec((1, H, E), lambda b, s: (s[b], 0, 0)),
            pl.BlockSpec((1, 1, E), lambda b, s: (s[b], 0, 0)),
            pl.BlockSpec((MBLK, 1), lambda b, s: (b, 0)),
        ],
        out_specs=pl.BlockSpec((MBLK, E), lambda b, s: (b, 0)),
        scratch_shapes=[pltpu.VMEM((H, E), jnp.bfloat16)],
    )
    ys = pl.pallas_call(
        _mm2_body,
        grid_spec=mm2_spec,
        out_shape=jax.ShapeDtypeStruct((PPAD, E), jnp.float32),
        compiler_params=pltpu.CompilerParams(
            dimension_semantics=("arbitrary",),
        ),
    )(bexp, hs, ew2, eb2.reshape(N, 1, E), wsort)

    combine_sc = pl.kernel(
        _combine_sc_body,
        out_type=jax.ShapeDtypeStruct((S, E), jnp.float32),
        mesh=_SC_MESH,
        scratch_types=[
            pltpu.VMEM((TOK_R,), jnp.int32),
            pltpu.VMEM((TOK_R,), jnp.int32),
            pltpu.VMEM((TOK_R // 2, E), jnp.float32),
            pltpu.VMEM((TOK_R // 2, E), jnp.float32),
            pltpu.SemaphoreType.DMA,
        ],
        compiler_params=_SC_PARAMS,
    )
    out = combine_sc(ys, posT)
    return out.reshape(1, S, E)


# speedup vs baseline: 1.6658x; 1.0406x over previous
import functools

import jax
import jax.numpy as jnp
from jax import lax
from jax.experimental import pallas as pl
from jax.experimental.pallas import tpu as pltpu
from jax.experimental.pallas import tpu_sc as plsc

S = 2048
E = 1024
N = 8
H = 4096
K = 2
P = S * K
MBLK = 256
PPAD = P + N * MBLK
NB = PPAD // MBLK
TBLK = 256
NC = 2
NS = 16
NW = NC * NS
LANES = 16
TOK_R = S // NW

_SC_MESH = plsc.VectorSubcoreMesh(core_axis_name="c", subcore_axis_name="s")
_SC_PARAMS = pltpu.CompilerParams(needs_layout_passes=False)


RG = 8


def _router_body(x_ref, rw1_ref, rb1_ref, rw2_ref, rb2_ref,
                 xbf_ref, wT_ref, posT_ref, bexp_ref,
                 ep_s, rk_s, w_s, carry_ref):
    t = pl.program_id(0)
    nblocks = RG

    @pl.when(t < RG)
    def _():
        xbf = x_ref[...].astype(jnp.bfloat16)
        xbf_ref[...] = xbf
        h = jnp.dot(xbf, rw1_ref[...].astype(jnp.bfloat16),
                    preferred_element_type=jnp.float32) + rb1_ref[...]
        h = jnp.maximum(h, 0.0).astype(jnp.bfloat16)
        s = jnp.dot(h, rw2_ref[...].astype(jnp.bfloat16),
                    preferred_element_type=jnp.float32) + rb2_ref[...]
        lane = lax.broadcasted_iota(jnp.int32, s.shape, 1)
        m1 = jnp.max(s, axis=1, keepdims=True)
        a1 = jnp.min(jnp.where(s == m1, lane, N), axis=1, keepdims=True)
        sm = jnp.where(lane == a1, -jnp.inf, s)
        m2 = jnp.max(sm, axis=1, keepdims=True)
        a2 = jnp.min(jnp.where(sm == m2, lane, N), axis=1, keepdims=True)
        e2 = jnp.exp(m2 - m1)
        w1 = 1.0 / (1.0 + e2)
        w2 = e2 / (1.0 + e2)
        w_s[pl.ds(t * TBLK, TBLK), :] = jnp.concatenate([w1, w2], axis=1)

        oh = ((lane == a1).astype(jnp.float32) +
              (lane == a2).astype(jnp.float32))
        r_i = lax.broadcasted_iota(jnp.int32, (TBLK, TBLK), 0)
        c_i = lax.broadcasted_iota(jnp.int32, (TBLK, TBLK), 1)
        tri = (c_i < r_i).astype(jnp.bfloat16)
        cum = jnp.dot(tri, oh.astype(jnp.bfloat16),
                      preferred_element_type=jnp.float32)

        @pl.when(t == 0)
        def _():
            carry_ref[...] = jnp.zeros_like(carry_ref)

        carry = carry_ref[...]
        cumg = cum + carry
        rank1 = jnp.sum(jnp.where(lane == a1, cumg, 0.0), axis=1,
                        keepdims=True)
        rank2 = jnp.sum(jnp.where(lane == a2, cumg, 0.0), axis=1,
                        keepdims=True)
        ep_s[pl.ds(t * TBLK, TBLK), :] = jnp.concatenate([a1, a2], axis=1)
        rk_s[pl.ds(t * TBLK, TBLK), :] = jnp.concatenate(
            [rank1, rank2], axis=1).astype(jnp.int32)
        carry_ref[...] = carry + jnp.sum(oh, axis=0, keepdims=True)

    @pl.when(t == nblocks - 1)
    def _():
        cnt = carry_ref[...]
        padded = jnp.floor((cnt + (MBLK - 1)) / MBLK) * MBLK
        ui = lax.broadcasted_iota(jnp.int32, (N, N), 0)
        uj = lax.broadcasted_iota(jnp.int32, (N, N), 1)
        offs = jnp.dot(padded.astype(jnp.bfloat16),
                       (ui < uj).astype(jnp.bfloat16),
                       preferred_element_type=jnp.float32)
        offsi = offs + padded
        lane8 = lax.broadcasted_iota(jnp.int32, (1, N), 1)
        ep = ep_s[...]
        rk = rk_s[...].astype(jnp.float32)
        posf = rk
        for e in range(N):
            off_e = jnp.sum(jnp.where(lane8 == e, offs, 0.0))
            posf = posf + jnp.where(ep == e, off_e, 0.0)
        posT_ref[...] = jnp.transpose(posf).astype(jnp.int32)
        wT_ref[...] = jnp.transpose(w_s[...])
        lane128 = lax.broadcasted_iota(jnp.int32, (1, 128), 1)
        row = (lane128 * MBLK).astype(jnp.float32)
        acc = jnp.zeros((1, 128), jnp.int32)
        for e in range(N):
            offi_e = jnp.sum(jnp.where(lane8 == e, offsi, 0.0))
            acc = acc + jnp.where(row >= offi_e, 1, 0)
        bexp_ref[...] = jnp.minimum(acc, N - 1).reshape(128)


def _expert_changed(bexp_ref, b):
    prev = bexp_ref[jnp.maximum(b - 1, 0)]
    return jnp.logical_or(b == 0, bexp_ref[b] != prev)


def _mm1_body(bexp_ref, posT_ref, wT_ref, xb_ref, ew1_ref, eb1_ref,
              h_ref, ws_ref, w1bf_s):
    b = pl.program_id(0)

    @pl.when(_expert_changed(bexp_ref, b))
    def _():
        w1bf_s[...] = ew1_ref[0].astype(jnp.bfloat16)

    slot = lax.broadcasted_iota(jnp.int32, (MBLK, S), 0) + b * MBLK
    p0 = posT_ref[0:1, :]
    p1 = posT_ref[1:2, :]
    sel0 = slot == p0
    sel1 = slot == p1
    perm = (sel0 | sel1).astype(jnp.bfloat16)
    xs = jnp.dot(perm, xb_ref[...],
                 preferred_element_type=jnp.float32).astype(jnp.bfloat16)
    ws_ref[...] = jnp.sum(jnp.where(sel0, wT_ref[0:1, :], 0.0) +
                          jnp.where(sel1, wT_ref[1:2, :], 0.0),
                          axis=1, keepdims=True)
    h = jnp.dot(xs, w1bf_s[...],
                preferred_element_type=jnp.float32) + eb1_ref[0]
    h_ref[...] = jnp.maximum(h, 0.0).astype(jnp.bfloat16)


def _mm2_body(bexp_ref, h_ref, ew2_ref, eb2_ref, ws_ref, o_ref, w2bf_s):
    b = pl.program_id(0)

    @pl.when(_expert_changed(bexp_ref, b))
    def _():
        w2bf_s[...] = ew2_ref[0].astype(jnp.bfloat16)

    y = jnp.dot(h_ref[...], w2bf_s[...],
                preferred_element_type=jnp.float32) + eb2_ref[0]
    o_ref[...] = y * ws_ref[...]


def _combine_sc_body(ys_hbm, posT_hbm, out_hbm, idx0_v, idx1_v, rows_a,
                     rows_b, sem):
    cid = lax.axis_index("c")
    sid = lax.axis_index("s")
    wid = sid * NC + cid
    tbase = wid * TOK_R
    pltpu.sync_copy(posT_hbm.at[0, pl.ds(tbase, TOK_R)], idx0_v)
    pltpu.sync_copy(posT_hbm.at[1, pl.ds(tbase, TOK_R)], idx1_v)
    half = TOK_R // 2
    for c in range(2):
        cp1 = pltpu.async_copy(
            ys_hbm.at[idx0_v.at[pl.ds(c * half, half)]], rows_a, sem)
        cp2 = pltpu.async_copy(
            ys_hbm.at[idx1_v.at[pl.ds(c * half, half)]], rows_b, sem)
        cp1.wait()
        cp2.wait()

        @pl.loop(0, half)
        def _(u):
            @pl.loop(0, E, step=LANES)
            def _(j):
                rows_a[u, pl.ds(j, LANES)] = (rows_a[u, pl.ds(j, LANES)] +
                                              rows_b[u, pl.ds(j, LANES)])

        pltpu.sync_copy(rows_a, out_hbm.at[pl.ds(tbase + c * half, half)])


@jax.jit
def kernel(inputs, rw1, rb1, rw2, rb2, ew1, eb1, ew2, eb2):
    x2 = inputs.reshape(S, E)

    xb, wT, posT, bexp = pl.pallas_call(
        _router_body,
        grid=(RG,),
        in_specs=[
            pl.BlockSpec((TBLK, E), lambda i: (i, 0)),
            pl.BlockSpec((E, E), lambda i: (0, 0)),
            pl.BlockSpec((E,), lambda i: (0,)),
            pl.BlockSpec((E, N), lambda i: (0, 0)),
            pl.BlockSpec((N,), lambda i: (0,)),
        ],
        out_specs=[
            pl.BlockSpec((TBLK, E), lambda i: (i, 0)),
            pl.BlockSpec((K, S), lambda i: (0, 0)),
            pl.BlockSpec((K, S), lambda i: (0, 0)),
            pl.BlockSpec((128,), lambda i: (0,)),
        ],
        out_shape=[
            jax.ShapeDtypeStruct((S, E), jnp.bfloat16),
            jax.ShapeDtypeStruct((K, S), jnp.float32),
            jax.ShapeDtypeStruct((K, S), jnp.int32),
            jax.ShapeDtypeStruct((128,), jnp.int32),
        ],
        scratch_shapes=[
            pltpu.VMEM((S, K), jnp.int32),
            pltpu.VMEM((S, K), jnp.int32),
            pltpu.VMEM((S, K), jnp.float32),
            pltpu.VMEM((1, N), jnp.float32),
        ],
    )(x2, rw1, rb1, rw2, rb2)

    mm1_spec = pltpu.PrefetchScalarGridSpec(
        num_scalar_prefetch=1,
        grid=(NB,),
        in_specs=[
            pl.BlockSpec((K, S), lambda b, s: (0, 0)),
            pl.BlockSpec((K, S), lambda b, s: (0, 0)),
            pl.BlockSpec((S, E), lambda b, s: (0, 0)),
            pl.BlockSpec((1, E, H), lambda b, s: (s[b], 0, 0)),
            pl.BlockSpec((1, 1, H), lambda b, s: (s[b], 0, 0)),
        ],
        out_specs=[
            pl.BlockSpec((MBLK, H), lambda b, s: (b, 0)),
            pl.BlockSpec((MBLK, 1), lambda b, s: (b, 0)),
        ],
        scratch_shapes=[pltpu.VMEM((E, H), jnp.bfloat16)],
    )
    hs, wsort = pl.pallas_call(
        _mm1_body,
        grid_spec=mm1_spec,
        out_shape=[
            jax.ShapeDtypeStruct((PPAD, H), jnp.bfloat16),
            jax.ShapeDtypeStruct((PPAD, 1), jnp.float32),
        ],
        compiler_params=pltpu.CompilerParams(
            dimension_semantics=("arbitrary",),
        ),
    )(bexp, posT, wT, xb, ew1, eb1.reshape(N, 1, H))

    mm2_spec = pltpu.PrefetchScalarGridSpec(
        num_scalar_prefetch=1,
        grid=(NB,),
        in_specs=[
            pl.BlockSpec((MBLK, H), lambda b, s: (b, 0)),
            pl.BlockSpec((1, H, E), lambda b, s: (s[b], 0, 0)),
            pl.BlockSpec((1, 1, E), lambda b, s: (s[b], 0, 0)),
            pl.BlockSpec((MBLK, 1), lambda b, s: (b, 0)),
        ],
        out_specs=pl.BlockSpec((MBLK, E), lambda b, s: (b, 0)),
        scratch_shapes=[pltpu.VMEM((H, E), jnp.bfloat16)],
    )
    ys = pl.pallas_call(
        _mm2_body,
        grid_spec=mm2_spec,
        out_shape=jax.ShapeDtypeStruct((PPAD, E), jnp.float32),
        compiler_params=pltpu.CompilerParams(
            dimension_semantics=("arbitrary",),
        ),
    )(bexp, hs, ew2, eb2.reshape(N, 1, E), wsort)

    combine_sc = pl.kernel(
        _combine_sc_body,
        out_type=jax.ShapeDtypeStruct((S, E), jnp.float32),
        mesh=_SC_MESH,
        scratch_types=[
            pltpu.VMEM((TOK_R,), jnp.int32),
            pltpu.VMEM((TOK_R,), jnp.int32),
            pltpu.VMEM((TOK_R // 2, E), jnp.float32),
            pltpu.VMEM((TOK_R // 2, E), jnp.float32),
            pltpu.SemaphoreType.DMA,
        ],
        compiler_params=_SC_PARAMS,
    )
    out = combine_sc(ys, posT)
    return out.reshape(1, S, E)
